# baseline (speedup 1.0000x reference)
import functools

import jax
import jax.numpy as jnp
from jax import lax
from jax.experimental import pallas as pl
from jax.experimental.pallas import tpu as pltpu
from jax.experimental.pallas import tpu_sc as plsc

N = 320000
C = 128
BQ = 4
NVOX = 40 * 40 * 40
NSEG = NVOX * BQ
SUB = 200
VPB = SUB // BQ
NBLK = NSEG // SUB
RC = 128
RSP = 1288
ZACC = SUB * C

_info = plsc.get_sparse_core_info()
NC, NS, L = _info.num_cores, _info.num_subcores, _info.num_lanes
W = NC * NS
NBW = NBLK // W
RSW = 56


def _seg_mean_body(feat_hbm, ranks_hbm, rst_hbm, out_hbm, fbuf, rbuf, acc,
                   cnt, rsv):
    w = lax.axis_index("s") * NC + lax.axis_index("c")
    pltpu.sync_copy(rst_hbm.at[pl.ds(pl.multiple_of(w * NBW, 8), RSW)], rsv)

    zero16 = jnp.zeros((L,), jnp.float32)
    for j in range(C // L):
        acc[pl.ds(ZACC + j * L, L)] = zero16

    def block_body(t, _):
        s_lo = (w * NBW + t) * SUB
        vlo = s_lo // BQ
        rs_v = rsv[pl.ds(t, L)]
        rs = rs_v[0]
        re = rs_v[1]
        base = lax.bitwise_and(rs, jnp.int32(-8))
        nchunks = jnp.where(re > rs, (re - base + (RC - 1)) // RC, 0)

        def czero_body(p, _):
            cnt[pl.ds(p * L, L)] = zero16
            return 0

        with jax.named_scope("ph_czero"):
            lax.fori_loop(0, SUB, czero_body, 0)

        def chunk_body(k, prev):
            nom = base + k * RC
            cstart = pl.multiple_of(jnp.minimum(nom, N - RC), 8)
            with jax.named_scope("ph_indma"):
                pltpu.sync_copy(
                    feat_hbm.at[pl.ds(pl.multiple_of(cstart * C, 128),
                                      RC * C)],
                    fbuf)
                pltpu.sync_copy(ranks_hbm.at[pl.ds(cstart, RC)],
                                rbuf.at[pl.ds(0, RC)])
            i_lo = jnp.maximum(rs, nom) - cstart
            i_hi = jnp.minimum(re, nom + RC) - cstart

            def row_body(i, prev_seg):
                seg = rbuf[pl.ds(i, L)][0]
                first = seg != prev_seg
                rel = seg - s_lo
                b = lax.bitwise_and(rel, 3)
                vl = lax.shift_right_logical(rel, 2)
                p = b * VPB + vl
                off = p * C
                src = jnp.where(first, ZACC, off)
                oc = cnt[pl.ds(p * L, L)]
                cnt[pl.ds(p * L, L)] = oc + 1.0
                fb = i * C
                for j in range(C // L):
                    f = fbuf[pl.ds(fb + j * L, L)]
                    a = acc[pl.ds(src + j * L, L)]
                    acc[pl.ds(off + j * L, L)] = a + f
                return seg

            with jax.named_scope("ph_rows"):
                return lax.fori_loop(i_lo, jnp.maximum(i_lo, i_hi), row_body,
                                     prev)

        lax.fori_loop(0, nchunks, chunk_body, jnp.int32(-1))

        one16 = jnp.full((L,), 1.0, jnp.float32)

        def div_body(p, _):
            c_v = cnt[pl.ds(p * L, L)]
            c_s = c_v[0]
            off = p * C
            src = jnp.where(c_s > 0.0, off, ZACC)
            inv_v = one16 / jnp.maximum(c_v, one16)
            for j in range(C // L):
                a = acc[pl.ds(src + j * L, L)]
                acc[pl.ds(off + j * L, L)] = a * inv_v
            return 0

        with jax.named_scope("ph_div"):
            lax.fori_loop(0, SUB, div_body, 0)

        with jax.named_scope("ph_outdma"):
            for b in range(BQ):
                pltpu.sync_copy(
                    acc.at[pl.ds(b * VPB * C, VPB * C)],
                    out_hbm.at[pl.ds(
                        pl.multiple_of((b * NVOX + vlo) * C, 128), VPB * C)])
        return 0

    lax.fori_loop(0, NBW, block_body, 0)


_seg_mean = functools.partial(
    pl.kernel,
    mesh=plsc.VectorSubcoreMesh(core_axis_name="c", subcore_axis_name="s"),
    out_type=jax.ShapeDtypeStruct((BQ * NVOX * C,), jnp.float32),
    scratch_types=[
        pltpu.VMEM((RC * C,), jnp.float32),
        pltpu.VMEM((RC + L,), jnp.int32),
        pltpu.VMEM((SUB * C + C,), jnp.float32),
        pltpu.VMEM((SUB * L,), jnp.float32),
        pltpu.VMEM((RSW,), jnp.int32),
    ],
)(_seg_mean_body)


def kernel(features, ranks):
    ranks_i32 = ranks.astype(jnp.int32)
    bounds = jnp.arange(NBLK + 1, dtype=jnp.int32) * SUB
    rstarts = jnp.searchsorted(ranks_i32, bounds, side="left",
                               method="scan_unrolled").astype(jnp.int32)
    rstarts = jnp.pad(rstarts, (0, RSP - (NBLK + 1)),
                      constant_values=jnp.int32(N))
    out = _seg_mean(features.reshape(-1), ranks_i32, rstarts)
    return out.reshape(BQ, 40, 40, 40, C)


# speedup vs baseline: 1.6668x; 1.6668x over previous
import functools

import jax
import jax.numpy as jnp
from jax import lax
from jax.experimental import pallas as pl
from jax.experimental.pallas import tpu as pltpu
from jax.experimental.pallas import tpu_sc as plsc

N = 320000
C = 128
NJ = C // 16
BQ = 4
NVOX = 40 * 40 * 40
NSEG = NVOX * BQ
SUB = 200
VPB = SUB // BQ
NBLK = NSEG // SUB
RC = 128
RSP = 1288
ZACC = SUB * C

_info = plsc.get_sparse_core_info()
NC, NS, L = _info.num_cores, _info.num_subcores, _info.num_lanes
W = NC * NS
NBW = NBLK // W
RSW = 56


def _seg_mean_body(feat_hbm, ranks_hbm, rst_hbm, out_hbm,
                   fbuf0, fbuf1, rbuf0, rbuf1, acc, cnt, rsv, sem0, sem1):
    fbufs = (fbuf0, fbuf1)
    rbufs = (rbuf0, rbuf1)
    sems = (sem0, sem1)
    w = lax.axis_index("s") * NC + lax.axis_index("c")
    pltpu.sync_copy(rst_hbm.at[pl.ds(pl.multiple_of(w * NBW, 8), RSW)], rsv)

    zero16 = jnp.zeros((L,), jnp.float32)
    one16 = jnp.full((L,), 1.0, jnp.float32)
    for j in range(NJ):
        acc[pl.ds(ZACC + j * L, L)] = zero16

    def block_body(t, _):
        s_lo = (w * NBW + t) * SUB
        vlo = s_lo // BQ
        rs_v = rsv[pl.ds(t, L)]
        rs = rs_v[0]
        re = rs_v[1]
        base = lax.bitwise_and(rs, jnp.int32(-8))
        nchunks = jnp.where(re > rs, (re - base + (RC - 1)) // RC, 0)

        @plsc.parallel_loop(0, SUB, unroll=8)
        def _czero(p):
            cnt[pl.ds(p * L, L)] = zero16

        def _cstart(k):
            return pl.multiple_of(jnp.minimum(base + k * RC, N - RC), 8)

        def _issue(k, bi):
            cstart = _cstart(k)
            pltpu.async_copy(
                feat_hbm.at[pl.ds(pl.multiple_of(cstart * C, 128), RC * C)],
                fbufs[bi], sems[bi])
            pltpu.async_copy(ranks_hbm.at[pl.ds(cstart, RC)],
                             rbufs[bi].at[pl.ds(0, RC)], sems[bi])

        @pl.when(nchunks > 0)
        def _():
            _issue(0, 0)

        def pair_body(kk, carry):
            for bi in range(2):
                k = kk * 2 + bi
                fbuf = fbufs[bi]
                rbuf = rbufs[bi]

                @pl.when(k + 1 < nchunks)
                def _():
                    _issue(k + 1, 1 - bi)

                @pl.when(k < nchunks)
                def _():
                    pltpu.make_async_copy(
                        feat_hbm.at[pl.ds(0, RC * C)], fbuf,
                        sems[bi]).wait()
                    pltpu.make_async_copy(
                        ranks_hbm.at[pl.ds(0, RC)],
                        rbuf.at[pl.ds(0, RC)], sems[bi]).wait()

                nom = base + k * RC
                cstart = _cstart(k)
                i_lo = jnp.maximum(rs, nom) - cstart
                i_hi = jnp.minimum(re, nom + RC) - cstart

                def row_body(i, rcarry):
                    prev, runlen, sums = rcarry
                    seg = rbuf[pl.ds(i, L)][0]
                    g = jnp.where(seg != prev, 0, 1)
                    runlen = runlen * g + 1
                    gv = jnp.full((L,), g, jnp.int32).astype(jnp.float32)
                    rel = seg - s_lo
                    b2 = lax.bitwise_and(rel, 3)
                    vl = lax.shift_right_logical(rel, 2)
                    p = b2 * VPB + vl
                    off = p * C
                    fb = i * C
                    new_sums = []
                    for j in range(NJ):
                        f = fbuf[pl.ds(fb + j * L, L)]
                        sj = sums[j] * gv + f
                        acc[pl.ds(off + j * L, L)] = sj
                        new_sums.append(sj)
                    cnt[pl.ds(p * L, L)] = jnp.full(
                        (L,), runlen, jnp.int32).astype(jnp.float32)
                    return (seg, runlen, tuple(new_sums))

                carry = lax.fori_loop(i_lo, jnp.maximum(i_lo, i_hi),
                                      row_body, carry)
            return carry

        init = (jnp.int32(-1), jnp.int32(0), tuple(zero16 for _ in range(NJ)))
        lax.fori_loop(0, (nchunks + 1) // 2, pair_body, init)

        @plsc.parallel_loop(0, SUB, unroll=4)
        def _div(p):
            c_v = cnt[pl.ds(p * L, L)]
            c_s = c_v[0]
            off = p * C
            src = jnp.where(c_s > 0.0, off, ZACC)
            inv_v = one16 / jnp.maximum(c_v, one16)
            for j in range(NJ):
                a = acc[pl.ds(src + j * L, L)]
                acc[pl.ds(off + j * L, L)] = a * inv_v

        for b in range(BQ):
            pltpu.sync_copy(
                acc.at[pl.ds(b * VPB * C, VPB * C)],
                out_hbm.at[pl.ds(
                    pl.multiple_of((b * NVOX + vlo) * C, 128), VPB * C)])
        return 0

    lax.fori_loop(0, NBW, block_body, 0)


_seg_mean = functools.partial(
    pl.kernel,
    mesh=plsc.VectorSubcoreMesh(core_axis_name="c", subcore_axis_name="s"),
    out_type=jax.ShapeDtypeStruct((BQ * NVOX * C,), jnp.float32),
    scratch_types=[
        pltpu.VMEM((RC * C,), jnp.float32),
        pltpu.VMEM((RC * C,), jnp.float32),
        pltpu.VMEM((RC + L,), jnp.int32),
        pltpu.VMEM((RC + L,), jnp.int32),
        pltpu.VMEM((SUB * C + C,), jnp.float32),
        pltpu.VMEM((SUB * L,), jnp.float32),
        pltpu.VMEM((RSW,), jnp.int32),
        pltpu.SemaphoreType.DMA,
        pltpu.SemaphoreType.DMA,
    ],
)(_seg_mean_body)


def kernel(features, ranks):
    ranks_i32 = ranks.astype(jnp.int32)
    bounds = jnp.arange(NBLK + 1, dtype=jnp.int32) * SUB
    rstarts = jnp.searchsorted(ranks_i32, bounds, side="left",
                               method="scan_unrolled").astype(jnp.int32)
    rstarts = jnp.pad(rstarts, (0, RSP - (NBLK + 1)),
                      constant_values=jnp.int32(N))
    out = _seg_mean(features.reshape(-1), ranks_i32, rstarts)
    return out.reshape(BQ, 40, 40, 40, C)
